# Initial kernel scaffold; baseline (speedup 1.0000x reference)
import jax
import jax.numpy as jnp
from jax.experimental import pallas as pl


def kernel(x, edge_index, W_in, b_in, Wl, bl, Wr, br, att, gat_bias, W_o1, b_o1, W_o, b_o):
    raise NotImplementedError("write your pallas kernel here")


# speedup vs baseline: 7.3595x; 7.3595x over previous
import functools

import jax
import jax.numpy as jnp
from jax import lax
from jax.experimental import pallas as pl
from jax.experimental.pallas import tpu as pltpu
from jax.experimental.pallas import tpu_sc as plsc

N = 10000
E = 160000
D_IN = 128
H = 64
W80 = 80
N_PAD = 10112
NC, NS = 2, 16
TILES = NC * NS
CHUNK = 128
E_PER_TILE = 5120
NCHUNK = E_PER_TILE // CHUNK
E_PAD = E_PER_TILE * TILES
ROWS_PER_TILE = N_PAD // NS
BLK = 200
NBLK = N // BLK
ECLAMP = 60.0

def _dot_t(a, b):
    return lax.dot_general(a.astype(jnp.bfloat16), b.astype(jnp.bfloat16),
                           (((1,), (1,)), ((), ())),
                           preferred_element_type=jnp.float32)


def _stage1_body(x_ref, win_ref, bin_ref, wl_ref, bl_ref, wr_ref, br_ref,
                 att_ref, hl_ref, hr_ref, selfc_ref):
    h = jnp.maximum(_dot_t(x_ref[...], win_ref[...]) + bin_ref[...], 0.0)
    hl = _dot_t(h, wl_ref[...]) + bl_ref[...]
    hr = _dot_t(h, wr_ref[...]) + br_ref[...]
    hl_ref[...] = hl
    hr_ref[...] = hr
    t = hl + hr
    t = jnp.maximum(t, 0.2 * t)
    e = lax.dot_general(t.astype(jnp.bfloat16),
                        att_ref[...].astype(jnp.bfloat16),
                        (((1,), (0,)), ((), ())),
                        preferred_element_type=jnp.float32)
    ex = jnp.exp(jnp.minimum(e, ECLAMP))
    selfc_ref[...] = jnp.concatenate(
        [ex * hl, jnp.broadcast_to(ex, (BLK, 16))], axis=1)


def _stage1(x, W_in, b_in, Wl, bl, Wr, br, att):
    return pl.pallas_call(
        _stage1_body,
        grid=(NBLK,),
        in_specs=[
            pl.BlockSpec((BLK, D_IN), lambda i: (i, 0)),
            pl.BlockSpec((H, D_IN), lambda i: (0, 0)),
            pl.BlockSpec((1, H), lambda i: (0, 0)),
            pl.BlockSpec((H, H), lambda i: (0, 0)),
            pl.BlockSpec((1, H), lambda i: (0, 0)),
            pl.BlockSpec((H, H), lambda i: (0, 0)),
            pl.BlockSpec((1, H), lambda i: (0, 0)),
            pl.BlockSpec((H, 1), lambda i: (0, 0)),
        ],
        out_specs=[
            pl.BlockSpec((BLK, H), lambda i: (i, 0)),
            pl.BlockSpec((BLK, H), lambda i: (i, 0)),
            pl.BlockSpec((BLK, W80), lambda i: (i, 0)),
        ],
        out_shape=[
            jax.ShapeDtypeStruct((N, H), jnp.float32),
            jax.ShapeDtypeStruct((N, H), jnp.float32),
            jax.ShapeDtypeStruct((N, W80), jnp.float32),
        ],
    )(x, W_in, b_in.reshape(1, H), Wl, bl.reshape(1, H), Wr, br.reshape(1, H),
      att.reshape(H, 1))


def _lane_sum(v):
    lanes = lax.iota(jnp.int32, 16)
    for m in (8, 4, 2, 1):
        v = v + v.at[lanes ^ m].get(mode="promise_in_bounds")
    return v


def _bf16_round(v):
    t = v * jnp.float32(65537.0)
    return t - (t - v)


def _sc_edges_body(hl_hbm, hr_hbm, att_hbm, src_hbm, dst_hbm, zeros_hbm,
                   out_hbm, src_v, dst_v, hl_rows, hr_rows, staged, att_v,
                   accum, sem1, sem2):
    c = lax.axis_index("c")
    s = lax.axis_index("s")
    wid = s * NC + c
    row0 = s * ROWS_PER_TILE
    pltpu.sync_copy(att_hbm, att_v)
    pltpu.sync_copy(zeros_hbm.at[pl.ds(row0, ROWS_PER_TILE)],
                    accum.at[pl.ds(row0, ROWS_PER_TILE)])
    for d in range(4):
        att_v[pl.ds(d * 16, 16)] = _bf16_round(att_v[pl.ds(d * 16, 16)])
    plsc.subcore_barrier()

    ebase = wid * E_PER_TILE

    def chunk_body(i, carry):
        off = ebase + i * CHUNK
        pltpu.sync_copy(src_hbm.at[pl.ds(off, CHUNK)], src_v)
        pltpu.sync_copy(dst_hbm.at[pl.ds(off, CHUNK)], dst_v)
        d1 = pltpu.async_copy(hl_hbm.at[src_v], hl_rows, sem1)
        d2 = pltpu.async_copy(hr_hbm.at[dst_v], hr_rows, sem2)
        d1.wait()
        d2.wait()

        def edge_body(j, carry2):
            acc = jnp.zeros((16,), jnp.float32)
            for d in range(4):
                a = hl_rows[j, pl.ds(d * 16, 16)]
                b = hr_rows[j, pl.ds(d * 16, 16)]
                t = a + b
                t = _bf16_round(jnp.maximum(t, 0.2 * t))
                acc = acc + t * att_v[pl.ds(d * 16, 16)]
            e = jnp.minimum(_lane_sum(acc), ECLAMP)
            ex = jnp.exp(e)
            for d in range(4):
                staged[j, pl.ds(d * 16, 16)] = ex * hl_rows[j, pl.ds(d * 16, 16)]
            staged[j, pl.ds(H, 16)] = ex
            return carry2

        lax.fori_loop(0, CHUNK, edge_body, 0, unroll=2)
        pltpu.sync_copy(staged, accum.at[dst_v], add=True)
        return carry

    lax.fori_loop(0, NCHUNK, chunk_body, 0)
    plsc.subcore_barrier()
    pltpu.sync_copy(accum.at[pl.ds(row0, ROWS_PER_TILE)],
                    out_hbm.at[c, pl.ds(row0, ROWS_PER_TILE)])


@functools.partial(
    pl.kernel,
    out_type=jax.ShapeDtypeStruct((NC, N_PAD, W80), jnp.float32),
    mesh=plsc.VectorSubcoreMesh(core_axis_name="c", subcore_axis_name="s"),
    scratch_types=[
        pltpu.VMEM((CHUNK,), jnp.int32),
        pltpu.VMEM((CHUNK,), jnp.int32),
        pltpu.VMEM((CHUNK, H), jnp.float32),
        pltpu.VMEM((CHUNK, H), jnp.float32),
        pltpu.VMEM((CHUNK, W80), jnp.float32),
        pltpu.VMEM((H,), jnp.float32),
        pltpu.VMEM_SHARED((N_PAD, W80), jnp.float32),
        pltpu.SemaphoreType.DMA,
        pltpu.SemaphoreType.DMA,
    ],
    compiler_params=pltpu.CompilerParams(use_tc_tiling_on_sc=False),
)
def _sc_edges(hl_hbm, hr_hbm, att_hbm, src_hbm, dst_hbm, zeros_hbm, out_hbm,
              src_v, dst_v, hl_rows, hr_rows, staged, att_v, accum, sem1,
              sem2):
    _sc_edges_body(hl_hbm, hr_hbm, att_hbm, src_hbm, dst_hbm, zeros_hbm,
                   out_hbm, src_v, dst_v, hl_rows, hr_rows, staged, att_v,
                   accum, sem1, sem2)


def _stage3a_body(part_ref, selfc_ref, gbias_ref, z_ref):
    tot = part_ref[0] + part_ref[1] + selfc_ref[...]
    num = tot[:, :H]
    den = tot[:, H:H + 1]
    gat = num / (den + 1e-16) + gbias_ref[...]
    z_ref[...] = jnp.maximum(gat, 0.2 * gat)


def _stage3a(partials, selfc, gat_bias):
    return pl.pallas_call(
        _stage3a_body,
        grid=(NBLK,),
        in_specs=[
            pl.BlockSpec((NC, BLK, W80), lambda i: (0, i, 0)),
            pl.BlockSpec((BLK, W80), lambda i: (i, 0)),
            pl.BlockSpec((1, H), lambda i: (0, 0)),
        ],
        out_specs=pl.BlockSpec((BLK, H), lambda i: (i, 0)),
        out_shape=jax.ShapeDtypeStruct((N, H), jnp.float32),
    )(partials, selfc, gat_bias.reshape(1, H))


def _stage3b_body(w1_ref, zf_ref, bo1_ref, wo_ref, bo_ref, y_ref, acc):
    i = pl.program_id(0)

    @pl.when(i == 0)
    def _init():
        acc[...] = jnp.zeros_like(acc)

    acc[...] += lax.dot_general(
        zf_ref[...].astype(jnp.bfloat16), w1_ref[...].astype(jnp.bfloat16),
        (((1,), (1,)), ((), ())), preferred_element_type=jnp.float32)

    @pl.when(i == NBLK - 1)
    def _fin():
        zo = jnp.maximum(acc[...] + bo1_ref[...], 0.0)
        prod = (zo.astype(jnp.bfloat16).astype(jnp.float32)
                * wo_ref[...].astype(jnp.bfloat16).astype(jnp.float32))
        y_ref[0, 0] = jnp.sum(prod) + bo_ref[0]


def _stage3b(W_o1, z_flat, b_o1, W_o, b_o):
    kblk = BLK * H
    return pl.pallas_call(
        _stage3b_body,
        grid=(NBLK,),
        in_specs=[
            pl.BlockSpec((H, kblk), lambda i: (0, i)),
            pl.BlockSpec((1, kblk), lambda i: (0, i)),
            pl.BlockSpec((1, H), lambda i: (0, 0)),
            pl.BlockSpec((1, H), lambda i: (0, 0)),
            pl.BlockSpec(memory_space=pltpu.MemorySpace.SMEM),
        ],
        out_specs=pl.BlockSpec(memory_space=pltpu.MemorySpace.SMEM),
        out_shape=jax.ShapeDtypeStruct((1, 1), jnp.float32),
        scratch_shapes=[pltpu.VMEM((1, H), jnp.float32)],
    )(W_o1, z_flat, b_o1.reshape(1, H), W_o, b_o)


def kernel(x, edge_index, W_in, b_in, Wl, bl, Wr, br, att, gat_bias, W_o1,
           b_o1, W_o, b_o):
    hl, hr, selfc = _stage1(x, W_in, b_in, Wl, bl, Wr, br, att)

    hl_t = jnp.pad(hl, ((0, N_PAD - N), (0, 0)))
    hr_t = jnp.pad(hr, ((0, N_PAD - N), (0, 0)))
    pad = jnp.full((E_PAD - E,), N, dtype=jnp.int32)
    src_p = jnp.concatenate([edge_index[0], pad])
    dst_p = jnp.concatenate([edge_index[1], pad])
    zeros = jnp.zeros((N_PAD, W80), jnp.float32)

    partials = _sc_edges(hl_t, hr_t, att, src_p, dst_p, zeros)

    z = _stage3a(partials, selfc, gat_bias)
    y = _stage3b(W_o1, z.reshape(1, N * H), b_o1, W_o, b_o)
    return y.reshape(1)


# speedup vs baseline: 9.3072x; 1.2646x over previous
import functools

import jax
import jax.numpy as jnp
from jax import lax
from jax.experimental import pallas as pl
from jax.experimental.pallas import tpu as pltpu
from jax.experimental.pallas import tpu_sc as plsc

N = 10000
E = 160000
D_IN = 128
H = 64
W80 = 80
N_PAD = 10112
NC, NS = 2, 16
TILES = NC * NS
CHUNK = 128
E_PER_TILE = 5120
NCHUNK = E_PER_TILE // CHUNK
E_PAD = E_PER_TILE * TILES
ROWS_PER_TILE = N_PAD // NS
BLK = 200
NBLK = N // BLK
ECLAMP = 60.0

def _dot_t(a, b):
    return lax.dot_general(a.astype(jnp.bfloat16), b.astype(jnp.bfloat16),
                           (((1,), (1,)), ((), ())),
                           preferred_element_type=jnp.float32)


def _stage1_body(x_ref, win_ref, bin_ref, wl_ref, bl_ref, wr_ref, br_ref,
                 att_ref, hl_ref, hr_ref, selfc_ref):
    h = jnp.maximum(_dot_t(x_ref[...], win_ref[...]) + bin_ref[...], 0.0)
    hl = _dot_t(h, wl_ref[...]) + bl_ref[...]
    hr = _dot_t(h, wr_ref[...]) + br_ref[...]
    hl_ref[...] = hl
    hr_ref[...] = hr
    t = hl + hr
    t = jnp.maximum(t, 0.2 * t)
    e = lax.dot_general(t.astype(jnp.bfloat16),
                        att_ref[...].astype(jnp.bfloat16),
                        (((1,), (0,)), ((), ())),
                        preferred_element_type=jnp.float32)
    ex = jnp.exp(jnp.minimum(e, ECLAMP))
    selfc_ref[...] = jnp.concatenate(
        [ex * hl, jnp.broadcast_to(ex, (BLK, 16))], axis=1)


def _stage1(x, W_in, b_in, Wl, bl, Wr, br, att):
    return pl.pallas_call(
        _stage1_body,
        grid=(NBLK,),
        in_specs=[
            pl.BlockSpec((BLK, D_IN), lambda i: (i, 0)),
            pl.BlockSpec((H, D_IN), lambda i: (0, 0)),
            pl.BlockSpec((1, H), lambda i: (0, 0)),
            pl.BlockSpec((H, H), lambda i: (0, 0)),
            pl.BlockSpec((1, H), lambda i: (0, 0)),
            pl.BlockSpec((H, H), lambda i: (0, 0)),
            pl.BlockSpec((1, H), lambda i: (0, 0)),
            pl.BlockSpec((H, 1), lambda i: (0, 0)),
        ],
        out_specs=[
            pl.BlockSpec((BLK, H), lambda i: (i, 0)),
            pl.BlockSpec((BLK, H), lambda i: (i, 0)),
            pl.BlockSpec((BLK, W80), lambda i: (i, 0)),
        ],
        out_shape=[
            jax.ShapeDtypeStruct((N, H), jnp.float32),
            jax.ShapeDtypeStruct((N, H), jnp.float32),
            jax.ShapeDtypeStruct((N, W80), jnp.float32),
        ],
    )(x, W_in, b_in.reshape(1, H), Wl, bl.reshape(1, H), Wr, br.reshape(1, H),
      att.reshape(H, 1))


def _lane_sum(v):
    lanes = lax.iota(jnp.int32, 16)
    for m in (8, 4, 2, 1):
        v = v + v.at[lanes ^ m].get(mode="promise_in_bounds")
    return v


def _bf16_round(v):
    t = v * jnp.float32(65537.0)
    return t - (t - v)


NPAIR = NCHUNK // 2


def _sc_edges_body(hl_hbm, hr_hbm, att_hbm, src_hbm, dst_hbm, zeros_hbm,
                   out_hbm, src_a, dst_a, src_b, dst_b, hl_a, hr_a, hl_b,
                   hr_b, staged, att_v, accum, sem_ia, sem_ib, sem_al,
                   sem_ar, sem_bl, sem_br):
    c = lax.axis_index("c")
    s = lax.axis_index("s")
    wid = s * NC + c
    row0 = s * ROWS_PER_TILE
    pltpu.sync_copy(att_hbm, att_v)
    pltpu.sync_copy(zeros_hbm.at[pl.ds(row0, ROWS_PER_TILE)],
                    accum.at[pl.ds(row0, ROWS_PER_TILE)])
    for d in range(4):
        att_v[pl.ds(d * 16, 16)] = _bf16_round(att_v[pl.ds(d * 16, 16)])
    plsc.subcore_barrier()

    ebase = wid * E_PER_TILE

    def fire_idx(g, sv, dv, sem):
        off = ebase + g * CHUNK
        pltpu.async_copy(src_hbm.at[pl.ds(off, CHUNK)], sv, sem)
        pltpu.async_copy(dst_hbm.at[pl.ds(off, CHUNK)], dv, sem)

    def wait_idx(sv, dv, sem):
        pltpu.make_async_copy(src_hbm.at[pl.ds(0, CHUNK)], sv, sem).wait()
        pltpu.make_async_copy(dst_hbm.at[pl.ds(0, CHUNK)], dv, sem).wait()

    def fire_rows(sv, dv, hlv, hrv, sl, sr):
        pltpu.async_copy(hl_hbm.at[sv], hlv, sl)
        pltpu.async_copy(hr_hbm.at[dv], hrv, sr)

    def wait_rows(sv, dv, hlv, hrv, sl, sr):
        pltpu.make_async_copy(hl_hbm.at[sv], hlv, sl).wait()
        pltpu.make_async_copy(hr_hbm.at[dv], hrv, sr).wait()

    def compute_chunk(hlv, hrv, dv):
        def edge_body(j, carry2):
            acc = jnp.zeros((16,), jnp.float32)
            for d in range(4):
                a = hlv[j, pl.ds(d * 16, 16)]
                b = hrv[j, pl.ds(d * 16, 16)]
                t = a + b
                t = _bf16_round(jnp.maximum(t, 0.2 * t))
                acc = acc + t * att_v[pl.ds(d * 16, 16)]
            e = jnp.minimum(_lane_sum(acc), ECLAMP)
            ex = jnp.exp(e)
            for d in range(4):
                staged[j, pl.ds(d * 16, 16)] = ex * hlv[j, pl.ds(d * 16, 16)]
            staged[j, pl.ds(H, 16)] = ex
            return carry2

        lax.fori_loop(0, CHUNK, edge_body, 0, unroll=4)
        pltpu.sync_copy(staged, accum.at[dv], add=True)

    fire_idx(0, src_a, dst_a, sem_ia)
    fire_idx(1, src_b, dst_b, sem_ib)
    wait_idx(src_a, dst_a, sem_ia)
    fire_rows(src_a, dst_a, hl_a, hr_a, sem_al, sem_ar)

    def pair_body(i, carry):
        g = 2 * i
        wait_idx(src_b, dst_b, sem_ib)
        fire_rows(src_b, dst_b, hl_b, hr_b, sem_bl, sem_br)
        wait_rows(src_a, dst_a, hl_a, hr_a, sem_al, sem_ar)
        compute_chunk(hl_a, hr_a, dst_a)

        @pl.when(i < NPAIR - 1)
        def _prefetch_a():
            fire_idx(g + 2, src_a, dst_a, sem_ia)

        @pl.when(i < NPAIR - 1)
        def _rows_a():
            wait_idx(src_a, dst_a, sem_ia)
            fire_rows(src_a, dst_a, hl_a, hr_a, sem_al, sem_ar)

        wait_rows(src_b, dst_b, hl_b, hr_b, sem_bl, sem_br)
        compute_chunk(hl_b, hr_b, dst_b)

        @pl.when(i < NPAIR - 1)
        def _prefetch_b():
            fire_idx(g + 3, src_b, dst_b, sem_ib)

        return carry

    lax.fori_loop(0, NPAIR, pair_body, 0)
    plsc.subcore_barrier()
    pltpu.sync_copy(accum.at[pl.ds(row0, ROWS_PER_TILE)],
                    out_hbm.at[c, pl.ds(row0, ROWS_PER_TILE)])


@functools.partial(
    pl.kernel,
    out_type=jax.ShapeDtypeStruct((NC, N_PAD, W80), jnp.float32),
    mesh=plsc.VectorSubcoreMesh(core_axis_name="c", subcore_axis_name="s"),
    scratch_types=[
        pltpu.VMEM((CHUNK,), jnp.int32),
        pltpu.VMEM((CHUNK,), jnp.int32),
        pltpu.VMEM((CHUNK,), jnp.int32),
        pltpu.VMEM((CHUNK,), jnp.int32),
        pltpu.VMEM((CHUNK, H), jnp.float32),
        pltpu.VMEM((CHUNK, H), jnp.float32),
        pltpu.VMEM((CHUNK, H), jnp.float32),
        pltpu.VMEM((CHUNK, H), jnp.float32),
        pltpu.VMEM((CHUNK, W80), jnp.float32),
        pltpu.VMEM((H,), jnp.float32),
        pltpu.VMEM_SHARED((N_PAD, W80), jnp.float32),
        pltpu.SemaphoreType.DMA,
        pltpu.SemaphoreType.DMA,
        pltpu.SemaphoreType.DMA,
        pltpu.SemaphoreType.DMA,
        pltpu.SemaphoreType.DMA,
        pltpu.SemaphoreType.DMA,
    ],
    compiler_params=pltpu.CompilerParams(use_tc_tiling_on_sc=False),
)
def _sc_edges(hl_hbm, hr_hbm, att_hbm, src_hbm, dst_hbm, zeros_hbm, out_hbm,
              src_a, dst_a, src_b, dst_b, hl_a, hr_a, hl_b, hr_b, staged,
              att_v, accum, sem_ia, sem_ib, sem_al, sem_ar, sem_bl, sem_br):
    _sc_edges_body(hl_hbm, hr_hbm, att_hbm, src_hbm, dst_hbm, zeros_hbm,
                   out_hbm, src_a, dst_a, src_b, dst_b, hl_a, hr_a, hl_b,
                   hr_b, staged, att_v, accum, sem_ia, sem_ib, sem_al,
                   sem_ar, sem_bl, sem_br)


def _stage3a_body(part_ref, selfc_ref, gbias_ref, z_ref):
    tot = part_ref[0] + part_ref[1] + selfc_ref[...]
    num = tot[:, :H]
    den = tot[:, H:H + 1]
    gat = num / (den + 1e-16) + gbias_ref[...]
    z_ref[...] = jnp.maximum(gat, 0.2 * gat)


def _stage3a(partials, selfc, gat_bias):
    return pl.pallas_call(
        _stage3a_body,
        grid=(NBLK,),
        in_specs=[
            pl.BlockSpec((NC, BLK, W80), lambda i: (0, i, 0)),
            pl.BlockSpec((BLK, W80), lambda i: (i, 0)),
            pl.BlockSpec((1, H), lambda i: (0, 0)),
        ],
        out_specs=pl.BlockSpec((BLK, H), lambda i: (i, 0)),
        out_shape=jax.ShapeDtypeStruct((N, H), jnp.float32),
    )(partials, selfc, gat_bias.reshape(1, H))


def _stage3b_body(w1_ref, zf_ref, bo1_ref, wo_ref, bo_ref, y_ref, acc):
    i = pl.program_id(0)

    @pl.when(i == 0)
    def _init():
        acc[...] = jnp.zeros_like(acc)

    acc[...] += lax.dot_general(
        zf_ref[...].astype(jnp.bfloat16), w1_ref[...].astype(jnp.bfloat16),
        (((1,), (1,)), ((), ())), preferred_element_type=jnp.float32)

    @pl.when(i == NBLK - 1)
    def _fin():
        zo = jnp.maximum(acc[...] + bo1_ref[...], 0.0)
        prod = (zo.astype(jnp.bfloat16).astype(jnp.float32)
                * wo_ref[...].astype(jnp.bfloat16).astype(jnp.float32))
        y_ref[0, 0] = jnp.sum(prod) + bo_ref[0]


def _stage3b(W_o1, z_flat, b_o1, W_o, b_o):
    kblk = BLK * H
    return pl.pallas_call(
        _stage3b_body,
        grid=(NBLK,),
        in_specs=[
            pl.BlockSpec((H, kblk), lambda i: (0, i)),
            pl.BlockSpec((1, kblk), lambda i: (0, i)),
            pl.BlockSpec((1, H), lambda i: (0, 0)),
            pl.BlockSpec((1, H), lambda i: (0, 0)),
            pl.BlockSpec(memory_space=pltpu.MemorySpace.SMEM),
        ],
        out_specs=pl.BlockSpec(memory_space=pltpu.MemorySpace.SMEM),
        out_shape=jax.ShapeDtypeStruct((1, 1), jnp.float32),
        scratch_shapes=[pltpu.VMEM((1, H), jnp.float32)],
    )(W_o1, z_flat, b_o1.reshape(1, H), W_o, b_o)


def kernel(x, edge_index, W_in, b_in, Wl, bl, Wr, br, att, gat_bias, W_o1,
           b_o1, W_o, b_o):
    hl, hr, selfc = _stage1(x, W_in, b_in, Wl, bl, Wr, br, att)

    hl_t = jnp.pad(hl, ((0, N_PAD - N), (0, 0)))
    hr_t = jnp.pad(hr, ((0, N_PAD - N), (0, 0)))
    pad = jnp.full((E_PAD - E,), N, dtype=jnp.int32)
    src_p = jnp.concatenate([edge_index[0], pad])
    dst_p = jnp.concatenate([edge_index[1], pad])
    zeros = jnp.zeros((N_PAD, W80), jnp.float32)

    partials = _sc_edges(hl_t, hr_t, att, src_p, dst_p, zeros)

    z = _stage3a(partials, selfc, gat_bias)
    y = _stage3b(W_o1, z.reshape(1, N * H), b_o1, W_o, b_o)
    return y.reshape(1)
